# Initial kernel scaffold; baseline (speedup 1.0000x reference)
import jax
import jax.numpy as jnp
from jax.experimental import pallas as pl


def kernel(drug_emb, entity_emb, relation_emb, edge_index, edge_type, gpu_id):
    raise NotImplementedError("write your pallas kernel here")


# speedup vs baseline: 1.4993x; 1.4993x over previous
import functools

import jax
import jax.numpy as jnp
from jax import lax
from jax.experimental import pallas as pl
from jax.experimental.pallas import tpu as pltpu
from jax.experimental.pallas import tpu_sc as plsc

CH = 128
CHH = CH // 2
L = 16
NC = 2
NS = 16
EB = 128
N_PAD = 10240
ROWS_PER_TILE = N_PAD // NS
CNT_W = 16
NREL_PAD = 16


def _make_sc_agg(nb, do_count):
    out_type = [jax.ShapeDtypeStruct((NC, N_PAD, CHH), jnp.float32)]
    if do_count:
        out_type.append(jax.ShapeDtypeStruct((N_PAD, CNT_W), jnp.float32))
    scratch = [
        pltpu.VMEM((nb, EB), jnp.int32),
        pltpu.VMEM((nb, EB), jnp.int32),
        pltpu.VMEM((nb, EB), jnp.int32),
        pltpu.VMEM((EB, CHH), jnp.float32),
        pltpu.VMEM((EB, CHH), jnp.float32),
        pltpu.VMEM_SHARED((N_PAD, CHH), jnp.float32),
        pltpu.SemaphoreType.DMA,
        pltpu.SemaphoreType.DMA,
    ]
    if do_count:
        scratch += [
            pltpu.VMEM((EB, CNT_W), jnp.float32),
            pltpu.VMEM_SHARED((N_PAD, CNT_W), jnp.float32),
        ]
    mesh = plsc.VectorSubcoreMesh(core_axis_name="c", subcore_axis_name="s")

    @functools.partial(pl.kernel, out_type=tuple(out_type), mesh=mesh,
                       scratch_types=scratch,
                       compiler_params=pltpu.CompilerParams(
                           use_tc_tiling_on_sc=False))
    def sc_agg(ent_hbm, rel_hbm, tail_hbm, type_hbm, head_hbm,
               zrow_hbm, zcnt_hbm, ones_hbm, *rest):
        if do_count:
            (part_hbm, cnt_hbm, tail_v, type_v, head_v, erow, rrow, acc,
             sem1, sem2, ones_v, cacc) = rest
        else:
            (part_hbm, tail_v, type_v, head_v, erow, rrow, acc,
             sem1, sem2) = rest
        c = lax.axis_index("c")
        s = lax.axis_index("s")
        pltpu.sync_copy(tail_hbm.at[s], tail_v)
        pltpu.sync_copy(type_hbm.at[s], type_v)
        pltpu.sync_copy(head_hbm.at[s], head_v)
        tile_rows = pl.ds(s * ROWS_PER_TILE, ROWS_PER_TILE)
        pltpu.sync_copy(zrow_hbm, acc.at[tile_rows])
        if do_count:
            pltpu.sync_copy(ones_hbm, ones_v)

            @pl.when(c == 0)
            def _():
                pltpu.sync_copy(zcnt_hbm, cacc.at[tile_rows])

        plsc.subcore_barrier()

        def batch_body(b, carry):
            cp1 = pltpu.async_copy(ent_hbm.at[c].at[tail_v.at[b]], erow, sem1)
            cp2 = pltpu.async_copy(rel_hbm.at[c].at[type_v.at[b]], rrow, sem2)
            cp1.wait()
            cp2.wait()

            def mul_body(e, carry2):
                for j in range(CHH // L):
                    sl = pl.ds(j * L, L)
                    erow[e, sl] = erow[e, sl] * rrow[e, sl]
                return carry2

            lax.fori_loop(0, EB, mul_body, 0, unroll=4)
            pltpu.sync_copy(erow, acc.at[head_v.at[b]], add=True)
            if do_count:
                @pl.when(c == 0)
                def _():
                    pltpu.sync_copy(ones_v, cacc.at[head_v.at[b]], add=True)

            return carry

        lax.fori_loop(0, nb, batch_body, 0)
        plsc.subcore_barrier()
        pltpu.sync_copy(acc.at[tile_rows], part_hbm.at[c, tile_rows])
        if do_count:
            @pl.when(c == 0)
            def _():
                pltpu.sync_copy(cacc.at[tile_rows], cnt_hbm.at[tile_rows])

    return sc_agg


TBLK = 1280


def _norm_block(part_ref, cnt_ref):
    ssum = jnp.concatenate([part_ref[0], part_ref[1]], axis=1)
    cnt = cnt_ref[:, 0:1]
    mean = ssum / jnp.maximum(cnt, 1.0)
    nrm = jnp.sqrt(jnp.sum(mean * mean, axis=1, keepdims=True))
    return mean / jnp.maximum(nrm, 1e-12)


def _tc_hop1(part, cnt, res_in):
    def body(part_ref, cnt_ref, res_ref, ent_next_ref, res_out_ref):
        normd = _norm_block(part_ref, cnt_ref)
        ent_next_ref[...] = normd
        res_out_ref[...] = res_ref[...] + normd

    grid = N_PAD // TBLK
    return pl.pallas_call(
        body,
        grid=(grid,),
        in_specs=[
            pl.BlockSpec((NC, TBLK, CHH), lambda i: (0, i, 0)),
            pl.BlockSpec((TBLK, CNT_W), lambda i: (i, 0)),
            pl.BlockSpec((TBLK, CH), lambda i: (i, 0)),
        ],
        out_specs=[
            pl.BlockSpec((TBLK, CH), lambda i: (i, 0)),
            pl.BlockSpec((TBLK, CH), lambda i: (i, 0)),
        ],
        out_shape=[
            jax.ShapeDtypeStruct((N_PAD, CH), jnp.float32),
            jax.ShapeDtypeStruct((N_PAD, CH), jnp.float32),
        ],
    )(part, cnt, res_in)


def _tc_hop2(part, cnt, res_in, drug0p, ent0p):
    def body(part_ref, cnt_ref, res_ref, drug0_ref, ent0_ref,
             res_out_ref, drug_ref):
        normd = _norm_block(part_ref, cnt_ref)
        r = res_ref[...] + normd
        res_out_ref[...] = r
        drug_ref[...] = drug0_ref[...] + (r - ent0_ref[...])

    grid = N_PAD // TBLK
    return pl.pallas_call(
        body,
        grid=(grid,),
        in_specs=[
            pl.BlockSpec((NC, TBLK, CHH), lambda i: (0, i, 0)),
            pl.BlockSpec((TBLK, CNT_W), lambda i: (i, 0)),
            pl.BlockSpec((TBLK, CH), lambda i: (i, 0)),
            pl.BlockSpec((TBLK, CH), lambda i: (i, 0)),
            pl.BlockSpec((TBLK, CH), lambda i: (i, 0)),
        ],
        out_specs=[
            pl.BlockSpec((TBLK, CH), lambda i: (i, 0)),
            pl.BlockSpec((TBLK, CH), lambda i: (i, 0)),
        ],
        out_shape=[
            jax.ShapeDtypeStruct((N_PAD, CH), jnp.float32),
            jax.ShapeDtypeStruct((N_PAD, CH), jnp.float32),
        ],
    )(part, cnt, res_in, drug0p, ent0p)


def _tc_rel(relp):
    def body(rel_ref, reln_ref, relres_ref):
        r = rel_ref[...]
        nrm = jnp.sqrt(jnp.sum(r * r, axis=1, keepdims=True))
        rn = r / jnp.maximum(nrm, 1e-12)
        reln_ref[...] = rn
        relres_ref[...] = r + 2.0 * rn

    return pl.pallas_call(
        body,
        out_shape=[
            jax.ShapeDtypeStruct((NREL_PAD, CH), jnp.float32),
            jax.ShapeDtypeStruct((NREL_PAD, CH), jnp.float32),
        ],
    )(relp)


def _split_ch(x):
    return jnp.stack([x[:, :CHH], x[:, CHH:]])


def kernel(drug_emb, entity_emb, relation_emb, edge_index, edge_type, gpu_id):
    n_ent, _ = entity_emb.shape
    n_drugs = drug_emb.shape[0]
    n_rel = relation_emb.shape[0]
    e = edge_type.shape[0]

    head = edge_index[0].astype(jnp.int32)
    tail = edge_index[1].astype(jnp.int32)
    etype = edge_type.astype(jnp.int32)

    nb = -(-e // (NS * EB))
    e_pad = NS * nb * EB
    pad = e_pad - e
    tail_p = jnp.concatenate([tail, jnp.zeros((pad,), jnp.int32)])
    etype_p = jnp.concatenate([etype, jnp.full((pad,), n_rel, jnp.int32)])
    head_p = jnp.concatenate([head, jnp.full((pad,), N_PAD - 8, jnp.int32)])
    tail_w = tail_p.reshape(NS, nb, EB)
    etype_w = etype_p.reshape(NS, nb, EB)
    head_w = head_p.reshape(NS, nb, EB)

    relp = jnp.zeros((NREL_PAD, CH), jnp.float32).at[:n_rel].set(relation_emb)
    zrow = jnp.zeros((ROWS_PER_TILE, CHH), jnp.float32)
    zcnt = jnp.zeros((ROWS_PER_TILE, CNT_W), jnp.float32)
    ones = jnp.ones((EB, CNT_W), jnp.float32)

    sc_agg1 = _make_sc_agg(nb, do_count=True)
    sc_agg2 = _make_sc_agg(nb, do_count=False)

    part1, cnt = sc_agg1(_split_ch(entity_emb), _split_ch(relp),
                         tail_w, etype_w, head_w, zrow, zcnt, ones)
    res0 = jnp.zeros((N_PAD, CH), jnp.float32).at[:n_ent].set(entity_emb)
    ent2, res1 = _tc_hop1(part1, cnt, res0)

    reln, relres = _tc_rel(relp)

    part2 = sc_agg2(_split_ch(ent2), _split_ch(reln),
                    tail_w, etype_w, head_w, zrow, zcnt, ones)
    if isinstance(part2, (tuple, list)):
        part2 = part2[0]
    drug0p = jnp.zeros((N_PAD, CH), jnp.float32).at[:n_drugs].set(drug_emb)
    res2, drug_full = _tc_hop2(part2, cnt, res1, drug0p, res0)

    entity_res = res2[:n_ent]
    drug_res = drug_full[:n_drugs]
    relation_res = relres[:n_rel]
    return (entity_res, drug_res, relation_res)


# speedup vs baseline: 3.1734x; 2.1166x over previous
import functools

import jax
import jax.numpy as jnp
from jax import lax
from jax.experimental import pallas as pl
from jax.experimental.pallas import tpu as pltpu
from jax.experimental.pallas import tpu_sc as plsc

CH = 128
CHH = CH // 2
L = 16
NC = 2
NS = 16
EB = 128
N_PAD = 10240
ROWS_PER_TILE = N_PAD // NS
CNT_W = 16
NREL_PAD = 16


RING = 3


def _make_sc_agg(nb):
    assert nb % (2 * RING) == 0
    IR = 2 * RING
    out_type = [
        jax.ShapeDtypeStruct((NC, N_PAD, CHH), jnp.float32),
        jax.ShapeDtypeStruct((N_PAD, CNT_W), jnp.float32),
    ]
    scratch = [
        pltpu.VMEM((IR, 3, EB), jnp.int32),
        pltpu.VMEM((RING, EB, CHH), jnp.float32),
        pltpu.VMEM((RING, EB, CHH), jnp.float32),
        pltpu.VMEM((NREL_PAD, CHH), jnp.float32),
        pltpu.VMEM((EB, CNT_W), jnp.float32),
        pltpu.VMEM_SHARED((N_PAD, CHH), jnp.float32),
        pltpu.VMEM_SHARED((N_PAD, CNT_W), jnp.float32),
    ]
    scratch += [pltpu.SemaphoreType.DMA] * RING
    scratch += [pltpu.SemaphoreType.DMA] * IR
    mesh = plsc.VectorSubcoreMesh(core_axis_name="c", subcore_axis_name="s")

    @functools.partial(pl.kernel, out_type=tuple(out_type), mesh=mesh,
                       scratch_types=scratch,
                       compiler_params=pltpu.CompilerParams(
                           use_tc_tiling_on_sc=False))
    def sc_agg(ent_hbm, rel_hbm, edata_hbm, zrow_hbm, zcnt_hbm, ones_hbm,
               *rest):
        (part_hbm, cnt_hbm, eslot, erow, msg, reltab, ones_v, acc, cacc,
         *sems) = rest
        sem_g = sems[:RING]
        sem_i = sems[RING:]
        c = lax.axis_index("c")
        s = lax.axis_index("s")
        pltpu.sync_copy(rel_hbm.at[c], reltab)
        tile_rows = pl.ds(s * ROWS_PER_TILE, ROWS_PER_TILE)
        pltpu.sync_copy(zrow_hbm, acc.at[tile_rows])
        pltpu.sync_copy(ones_hbm, ones_v)

        @pl.when(c == 0)
        def _():
            pltpu.sync_copy(zcnt_hbm, cacc.at[tile_rows])

        plsc.subcore_barrier()

        def fire_idx(u, b):
            pltpu.async_copy(edata_hbm.at[s, b], eslot.at[u], sem_i[u])

        def wait_idx(u):
            pltpu.make_async_copy(edata_hbm.at[s, 0], eslot.at[u],
                                  sem_i[u]).wait()

        def fire_row(k, u, b):
            pltpu.async_copy(ent_hbm.at[c].at[eslot.at[u, 0]], erow.at[k],
                             sem_g[k])

        def wait_row(k):
            pltpu.make_async_copy(ent_hbm.at[c].at[eslot.at[0, 0]],
                                  erow.at[k], sem_g[k]).wait()

        for u in range(IR):
            fire_idx(u, u)
        for k in range(RING):
            wait_idx(k)
            fire_row(k, k, k)

        def group_body(g, carry):
            for u in range(IR):
                k = u % RING
                b = g * IR + u
                wait_row(k)

                def mul_body(g2, carry2):
                    tvec = eslot[u, 1, pl.ds(g2 * L, L)]
                    for i in range(L):
                        e = g2 * L + i
                        t = tvec[i]
                        for j in range(CHH // L):
                            sl = pl.ds(j * L, L)
                            msg[k, e, sl] = erow[k, e, sl] * reltab[t, sl]
                    return carry2

                lax.fori_loop(0, EB // L, mul_body, 0)

                pltpu.sync_copy(msg.at[k], acc.at[eslot.at[u, 2]], add=True)

                @pl.when(c == 0)
                def _():
                    pltpu.sync_copy(ones_v, cacc.at[eslot.at[u, 2]],
                                    add=True)

                @pl.when(b + IR < nb)
                def _():
                    fire_idx(u, b + IR)

                @pl.when(b + RING < nb)
                def _():
                    u2 = (u + RING) % IR
                    wait_idx(u2)
                    fire_row(k, u2, b + RING)
            return carry

        lax.fori_loop(0, nb // IR, group_body, 0)
        plsc.subcore_barrier()
        pltpu.sync_copy(acc.at[tile_rows], part_hbm.at[c, tile_rows])

        @pl.when(c == 0)
        def _():
            pltpu.sync_copy(cacc.at[tile_rows], cnt_hbm.at[tile_rows])

    return sc_agg


TBLK = 1280


def _norm_block(part_ref, cnt_ref):
    ssum = jnp.concatenate([part_ref[0], part_ref[1]], axis=1)
    cnt = cnt_ref[:, 0:1]
    mean = ssum / jnp.maximum(cnt, 1.0)
    nrm = jnp.sqrt(jnp.sum(mean * mean, axis=1, keepdims=True))
    return mean / jnp.maximum(nrm, 1e-12)


def _tc_hop1(part, cnt, res_in):
    def body(part_ref, cnt_ref, res_ref, ent_next_ref, res_out_ref):
        normd = _norm_block(part_ref, cnt_ref)
        ent_next_ref[...] = normd
        res_out_ref[...] = res_ref[...] + normd

    grid = N_PAD // TBLK
    return pl.pallas_call(
        body,
        grid=(grid,),
        in_specs=[
            pl.BlockSpec((NC, TBLK, CHH), lambda i: (0, i, 0)),
            pl.BlockSpec((TBLK, CNT_W), lambda i: (i, 0)),
            pl.BlockSpec((TBLK, CH), lambda i: (i, 0)),
        ],
        out_specs=[
            pl.BlockSpec((TBLK, CH), lambda i: (i, 0)),
            pl.BlockSpec((TBLK, CH), lambda i: (i, 0)),
        ],
        out_shape=[
            jax.ShapeDtypeStruct((N_PAD, CH), jnp.float32),
            jax.ShapeDtypeStruct((N_PAD, CH), jnp.float32),
        ],
    )(part, cnt, res_in)


def _tc_drug(res2, drug0p, ent0p):
    def body(res_ref, drug0_ref, ent0_ref, drug_ref):
        drug_ref[...] = drug0_ref[...] + (res_ref[...] - ent0_ref[...])

    grid = N_PAD // TBLK
    return pl.pallas_call(
        body,
        grid=(grid,),
        in_specs=[
            pl.BlockSpec((TBLK, CH), lambda i: (i, 0)),
            pl.BlockSpec((TBLK, CH), lambda i: (i, 0)),
            pl.BlockSpec((TBLK, CH), lambda i: (i, 0)),
        ],
        out_specs=[
            pl.BlockSpec((TBLK, CH), lambda i: (i, 0)),
        ],
        out_shape=[
            jax.ShapeDtypeStruct((N_PAD, CH), jnp.float32),
        ],
    )(res2, drug0p, ent0p)[0]


def _tc_rel(relp):
    def body(rel_ref, reln_ref, relres_ref):
        r = rel_ref[...]
        nrm = jnp.sqrt(jnp.sum(r * r, axis=1, keepdims=True))
        rn = r / jnp.maximum(nrm, 1e-12)
        reln_ref[...] = rn
        relres_ref[...] = r + 2.0 * rn

    return pl.pallas_call(
        body,
        out_shape=[
            jax.ShapeDtypeStruct((NREL_PAD, CH), jnp.float32),
            jax.ShapeDtypeStruct((NREL_PAD, CH), jnp.float32),
        ],
    )(relp)


def _split_ch(x):
    return jnp.stack([x[:, :CHH], x[:, CHH:]])


def kernel(drug_emb, entity_emb, relation_emb, edge_index, edge_type, gpu_id):
    n_ent, _ = entity_emb.shape
    n_drugs = drug_emb.shape[0]
    n_rel = relation_emb.shape[0]
    e = edge_type.shape[0]

    head = edge_index[0].astype(jnp.int32)
    tail = edge_index[1].astype(jnp.int32)
    etype = edge_type.astype(jnp.int32)

    nb = 2 * RING * -(-e // (NS * EB * 2 * RING))
    e_pad = NS * nb * EB
    pad = e_pad - e
    tail_p = jnp.concatenate([tail, jnp.zeros((pad,), jnp.int32)])
    etype_p = jnp.concatenate([etype, jnp.full((pad,), n_rel, jnp.int32)])
    head_p = jnp.concatenate([head, jnp.full((pad,), N_PAD - 8, jnp.int32)])
    edata = jnp.stack([tail_p.reshape(NS, nb, EB),
                       etype_p.reshape(NS, nb, EB),
                       head_p.reshape(NS, nb, EB)], axis=2)

    relp = jnp.zeros((NREL_PAD, CH), jnp.float32).at[:n_rel].set(relation_emb)
    zrow = jnp.zeros((ROWS_PER_TILE, CHH), jnp.float32)
    zcnt = jnp.zeros((ROWS_PER_TILE, CNT_W), jnp.float32)
    ones = jnp.ones((EB, CNT_W), jnp.float32)

    sc_agg = _make_sc_agg(nb)

    reln, relres = _tc_rel(relp)

    res0 = jnp.zeros((N_PAD, CH), jnp.float32).at[:n_ent].set(entity_emb)

    def hop(carry, rel_tab):
        ent_tab, res = carry
        part, cnt = sc_agg(ent_tab, rel_tab, edata, zrow, zcnt, ones)
        ent_next, res_next = _tc_hop1(part, cnt, res)
        return (_split_ch(ent_next), res_next), None

    rel_tabs = jnp.stack([_split_ch(relp), _split_ch(reln)])
    (_, res2), _ = lax.scan(hop, (_split_ch(res0), res0), rel_tabs)

    drug0p = jnp.zeros((N_PAD, CH), jnp.float32).at[:n_drugs].set(drug_emb)
    drug_full = _tc_drug(res2, drug0p, res0)

    entity_res = res2[:n_ent]
    drug_res = drug_full[:n_drugs]
    relation_res = relres[:n_rel]
    return (entity_res, drug_res, relation_res)
